# baseline (speedup 1.0000x reference)
import jax
import jax.numpy as jnp
from jax import lax
from jax.experimental import pallas as pl
from jax.experimental.pallas import tpu as pltpu
from jax.experimental.pallas import tpu_sc as plsc

N_LAYERS = 9
M = 2048
N = 2048
K_MATCH = 1024
K_UNA = 512
K_UNB = 512

NLANE = 16
HALF_M = K_MATCH // 2
HALF_A = K_UNA // 2
HALF_B = K_UNB // 2
MB = 16
N_MB = HALF_M // MB
STRIP = 2056

W_MATCH = -1.0 / K_MATCH
W_UNA = -0.5 / K_UNA
W_UNB = -0.5 / K_UNB

_mesh = plsc.VectorSubcoreMesh(
    core_axis_name="c", subcore_axis_name="s", num_cores=2, num_subcores=16
)

_scratch_types = [
    pltpu.VMEM((HALF_M,), jnp.int32),
    pltpu.VMEM((HALF_M,), jnp.int32),
    pltpu.VMEM((HALF_A,), jnp.int32),
    pltpu.VMEM((HALF_B,), jnp.int32),
    pltpu.VMEM((MB, 8, 128), jnp.float32),
    pltpu.VMEM((STRIP,), jnp.float32),
    pltpu.VMEM((STRIP,), jnp.float32),
    pltpu.VMEM((NLANE,), jnp.float32),
    pltpu.VMEM((NLANE, NLANE), jnp.float32),
    pltpu.SemaphoreType.DMA,
]


def _sc_loss_body(scores_hbm, mr_hbm, mc_hbm, ua_hbm, ub_hbm, cola_hbm,
                  rowb_hbm, out_hbm, stage_hbm, *scratch):
    (mr_v, mc_v, ua_v, ub_v, tile_v, cola_v, rowb_v, vec_v, part_v, sem) = scratch
    c = lax.axis_index("c")
    s = lax.axis_index("s")
    lanes = lax.iota(jnp.int32, NLANE)

    @pl.when(s < N_LAYERS)
    def _worker():
        l = s
        h = c
        pltpu.sync_copy(mr_hbm.at[pl.ds(h * HALF_M, HALF_M)], mr_v)
        pltpu.sync_copy(mc_hbm.at[pl.ds(h * HALF_M, HALF_M)], mc_v)
        pltpu.sync_copy(ua_hbm.at[pl.ds(h * HALF_A, HALF_A)], ua_v)
        pltpu.sync_copy(ub_hbm.at[pl.ds(h * HALF_B, HALF_B)], ub_v)
        pltpu.sync_copy(cola_hbm.at[pl.ds(l * STRIP, STRIP)], cola_v)
        pltpu.sync_copy(rowb_hbm.at[pl.ds(l * STRIP, STRIP)], rowb_v)

        def match_batch(j, acc):
            base = j * MB
            mr16 = mr_v[pl.ds(base, MB)]
            mc16 = mc_v[pl.ds(base, MB)]
            copies = []
            for i in range(MB):
                r = mr16[i]
                cc = mc16[i]
                cp = pltpu.async_copy(
                    scores_hbm.at[l, pl.ds((r // 8) * 8, 8),
                                  pl.ds((cc // 128) * 128, 128)],
                    tile_v.at[i],
                    sem,
                )
                copies.append(cp)
            for cp in copies:
                cp.wait()
            for i in range(MB):
                r = mr16[i]
                cc = mc16[i]
                v16 = tile_v[i, r % 8, pl.ds(((cc % 128) // 16) * 16, NLANE)]
                sel = jnp.where(lanes == cc % NLANE, v16, 0.0)
                acc = acc + sel * jnp.float32(W_MATCH)
            return acc

        acc = lax.fori_loop(0, N_MB, match_batch,
                            jnp.zeros((NLANE,), jnp.float32))

        def strip_batch(strip_ref, idx_ref, w):
            def body(j, acc):
                base = j * MB
                x16 = idx_ref[pl.ds(base, MB)]
                for i in range(MB):
                    x = x16[i]
                    v16 = strip_ref[pl.ds((x // NLANE) * NLANE, NLANE)]
                    sel = jnp.where(lanes == x % NLANE, v16, 0.0)
                    acc = acc + sel * jnp.float32(w)
                return acc
            return body

        acc = lax.fori_loop(0, HALF_A // MB, strip_batch(cola_v, ua_v, W_UNA), acc)
        acc = lax.fori_loop(0, HALF_B // MB, strip_batch(rowb_v, ub_v, W_UNB), acc)

        vec_v[...] = acc
        pltpu.sync_copy(vec_v, stage_hbm.at[c, s])

    plsc.subcore_barrier()

    @pl.when(s == 0)
    def _finalize():
        pltpu.sync_copy(stage_hbm.at[c], part_v)

        def allsum(v):
            for sh in (1, 2, 4, 8):
                v = v + v.at[lanes ^ sh].get(mode="promise_in_bounds")
            return v

        out = jnp.zeros((NLANE,), jnp.float32)
        total = jnp.zeros((NLANE,), jnp.float32)
        for l in range(N_LAYERS):
            loss_l = allsum(part_v[l, :])
            out = jnp.where(lanes == l, loss_l, out)
            total = total + loss_l
        total = total * jnp.float32(1.0 / N_LAYERS)
        out = jnp.where(lanes == N_LAYERS, total, out)
        vec_v[...] = out
        pltpu.sync_copy(vec_v, out_hbm.at[c])


_sc_loss = pl.kernel(
    _sc_loss_body,
    out_type=(
        jax.ShapeDtypeStruct((2, NLANE), jnp.float32),
        jax.ShapeDtypeStruct((2, NLANE, NLANE), jnp.float32),
    ),
    mesh=_mesh,
    scratch_types=_scratch_types,
)


def kernel(scores_per_layer, matches, unmatchable_A, unmatchable_B):
    mr = matches[:, 0].astype(jnp.int32)
    mc = matches[:, 1].astype(jnp.int32)
    cola = jnp.pad(scores_per_layer[:, :, -1], ((0, 0), (0, STRIP - (M + 1)))).reshape(-1)
    rowb = jnp.pad(scores_per_layer[:, -1, :], ((0, 0), (0, STRIP - (N + 1)))).reshape(-1)
    out2, _ = _sc_loss(
        scores_per_layer,
        mr,
        mc,
        unmatchable_A.astype(jnp.int32),
        unmatchable_B.astype(jnp.int32),
        cola,
        rowb,
    )
    return (out2[0] + out2[1])[: N_LAYERS + 1]


# speedup vs baseline: 1.1278x; 1.1278x over previous
import jax
import jax.numpy as jnp
from jax import lax
from jax.experimental import pallas as pl
from jax.experimental.pallas import tpu as pltpu
from jax.experimental.pallas import tpu_sc as plsc

N_LAYERS = 9
M = 2048
N = 2048
K_MATCH = 1024
K_UNA = 512
K_UNB = 512

NLANE = 16
HALF_M = K_MATCH // 2
HALF_A = K_UNA // 2
HALF_B = K_UNB // 2
MB = 32
N_MB = HALF_M // MB
STRIP = 2056

W_MATCH = -1.0 / K_MATCH
W_UNA = -0.5 / K_UNA
W_UNB = -0.5 / K_UNB

_mesh = plsc.VectorSubcoreMesh(
    core_axis_name="c", subcore_axis_name="s", num_cores=2, num_subcores=16
)

_scratch_types = [
    pltpu.VMEM((HALF_M,), jnp.int32),
    pltpu.VMEM((HALF_M,), jnp.int32),
    pltpu.VMEM((HALF_A,), jnp.int32),
    pltpu.VMEM((HALF_B,), jnp.int32),
    pltpu.VMEM((MB, 8, 128), jnp.float32),
    pltpu.VMEM((STRIP,), jnp.float32),
    pltpu.VMEM((STRIP,), jnp.float32),
    pltpu.VMEM((NLANE,), jnp.float32),
    pltpu.VMEM((NLANE, NLANE), jnp.float32),
    pltpu.SemaphoreType.DMA,
]


def _sc_loss_body(scores_hbm, mr_hbm, mc_hbm, ua_hbm, ub_hbm, cola_hbm,
                  rowb_hbm, out_hbm, stage_hbm, *scratch):
    (mr_v, mc_v, ua_v, ub_v, tile_v, cola_v, rowb_v, vec_v, part_v, sem) = scratch
    c = lax.axis_index("c")
    s = lax.axis_index("s")
    lanes = lax.iota(jnp.int32, NLANE)

    @pl.when(s < N_LAYERS)
    def _worker():
        l = s
        h = c
        pltpu.sync_copy(mr_hbm.at[pl.ds(h * HALF_M, HALF_M)], mr_v)
        pltpu.sync_copy(mc_hbm.at[pl.ds(h * HALF_M, HALF_M)], mc_v)
        pltpu.sync_copy(ua_hbm.at[pl.ds(h * HALF_A, HALF_A)], ua_v)
        pltpu.sync_copy(ub_hbm.at[pl.ds(h * HALF_B, HALF_B)], ub_v)
        pltpu.sync_copy(cola_hbm.at[pl.ds(l * STRIP, STRIP)], cola_v)
        pltpu.sync_copy(rowb_hbm.at[pl.ds(l * STRIP, STRIP)], rowb_v)

        def match_batch(j, acc):
            base = j * MB
            chunks = [
                (mr_v[pl.ds(base + g * NLANE, NLANE)],
                 mc_v[pl.ds(base + g * NLANE, NLANE)])
                for g in range(MB // NLANE)
            ]
            copies = []
            for g, (mr16, mc16) in enumerate(chunks):
                for i in range(NLANE):
                    r = mr16[i]
                    cc = mc16[i]
                    cp = pltpu.async_copy(
                        scores_hbm.at[l, pl.ds((r // 8) * 8, 8),
                                      pl.ds((cc // 128) * 128, 128)],
                        tile_v.at[g * NLANE + i],
                        sem,
                    )
                    copies.append(cp)
            for cp in copies:
                cp.wait()
            for g, (mr16, mc16) in enumerate(chunks):
                for i in range(NLANE):
                    r = mr16[i]
                    cc = mc16[i]
                    v16 = tile_v[g * NLANE + i, r % 8,
                                 pl.ds(((cc % 128) // 16) * 16, NLANE)]
                    sel = jnp.where(lanes == cc % NLANE, v16, 0.0)
                    acc = acc + sel * jnp.float32(W_MATCH)
            return acc

        acc = lax.fori_loop(0, N_MB, match_batch,
                            jnp.zeros((NLANE,), jnp.float32))

        def strip_batch(strip_ref, idx_ref, w):
            def body(j, acc):
                base = j * NLANE
                x16 = idx_ref[pl.ds(base, NLANE)]
                for i in range(NLANE):
                    x = x16[i]
                    v16 = strip_ref[pl.ds((x // NLANE) * NLANE, NLANE)]
                    sel = jnp.where(lanes == x % NLANE, v16, 0.0)
                    acc = acc + sel * jnp.float32(w)
                return acc
            return body

        acc = lax.fori_loop(0, HALF_A // NLANE, strip_batch(cola_v, ua_v, W_UNA), acc)
        acc = lax.fori_loop(0, HALF_B // NLANE, strip_batch(rowb_v, ub_v, W_UNB), acc)

        vec_v[...] = acc
        pltpu.sync_copy(vec_v, stage_hbm.at[c, s])

    plsc.subcore_barrier()

    @pl.when(s == 0)
    def _finalize():
        pltpu.sync_copy(stage_hbm.at[c], part_v)

        def allsum(v):
            for sh in (1, 2, 4, 8):
                v = v + v.at[lanes ^ sh].get(mode="promise_in_bounds")
            return v

        out = jnp.zeros((NLANE,), jnp.float32)
        total = jnp.zeros((NLANE,), jnp.float32)
        for l in range(N_LAYERS):
            loss_l = allsum(part_v[l, :])
            out = jnp.where(lanes == l, loss_l, out)
            total = total + loss_l
        total = total * jnp.float32(1.0 / N_LAYERS)
        out = jnp.where(lanes == N_LAYERS, total, out)
        vec_v[...] = out
        pltpu.sync_copy(vec_v, out_hbm.at[c])


_sc_loss = pl.kernel(
    _sc_loss_body,
    out_type=(
        jax.ShapeDtypeStruct((2, NLANE), jnp.float32),
        jax.ShapeDtypeStruct((2, NLANE, NLANE), jnp.float32),
    ),
    mesh=_mesh,
    scratch_types=_scratch_types,
)


def kernel(scores_per_layer, matches, unmatchable_A, unmatchable_B):
    mr = matches[:, 0].astype(jnp.int32)
    mc = matches[:, 1].astype(jnp.int32)
    cola = jnp.pad(scores_per_layer[:, :, -1], ((0, 0), (0, STRIP - (M + 1)))).reshape(-1)
    rowb = jnp.pad(scores_per_layer[:, -1, :], ((0, 0), (0, STRIP - (N + 1)))).reshape(-1)
    out2, _ = _sc_loss(
        scores_per_layer,
        mr,
        mc,
        unmatchable_A.astype(jnp.int32),
        unmatchable_B.astype(jnp.int32),
        cola,
        rowb,
    )
    return (out2[0] + out2[1])[: N_LAYERS + 1]
